# Initial kernel scaffold; baseline (speedup 1.0000x reference)
import jax
import jax.numpy as jnp
from jax.experimental import pallas as pl


def kernel(x, edge_index, params):
    raise NotImplementedError("write your pallas kernel here")


# speedup vs baseline: 3.0285x; 3.0285x over previous
import functools

import jax
import jax.numpy as jnp
from jax.experimental import pallas as pl
from jax.experimental.pallas import tpu as pltpu

_HEADS = 8
_D = 128


def _lin_body(x_ref, w_ref, b_ref, o_ref, *, act):
    y = jnp.dot(x_ref[...], w_ref[...], preferred_element_type=jnp.float32)
    y = y + b_ref[...]
    if act == "relu":
        y = jnp.maximum(y, 0.0)
    o_ref[...] = y


def _linear(x, w, b, act=None, rb=None):
    n, k = x.shape
    m = w.shape[1]
    if rb is None:
        rb = n if n <= 2000 else 1000
    assert n % rb == 0
    grid = (n // rb,)
    return pl.pallas_call(
        functools.partial(_lin_body, act=act),
        grid=grid,
        in_specs=[
            pl.BlockSpec((rb, k), lambda i: (i, 0)),
            pl.BlockSpec((k, m), lambda i: (0, 0)),
            pl.BlockSpec((1, m), lambda i: (0, 0)),
        ],
        out_specs=pl.BlockSpec((rb, m), lambda i: (i, 0)),
        out_shape=jax.ShapeDtypeStruct((n, m), jnp.float32),
    )(x, w, b.reshape(1, m))


def _mlp2_body(x_ref, w1_ref, b1_ref, w2_ref, b2_ref, o_ref, *, logsm):
    mid = jnp.dot(x_ref[...], w1_ref[...], preferred_element_type=jnp.float32)
    mid = jnp.maximum(mid + b1_ref[...], 0.0)
    y = jnp.dot(mid, w2_ref[...], preferred_element_type=jnp.float32)
    y = y + b2_ref[...]
    if logsm:
        y = y - jnp.max(y, axis=1, keepdims=True)
        y = y - jnp.log(jnp.sum(jnp.exp(y), axis=1, keepdims=True))
    o_ref[...] = y


def _mlp2(x, w1, b1, w2, b2, logsm=False, rb=None):
    n, k = x.shape
    f = w1.shape[1]
    m = w2.shape[1]
    if rb is None:
        rb = n if n <= 2000 else 1000
    assert n % rb == 0
    grid = (n // rb,)
    return pl.pallas_call(
        functools.partial(_mlp2_body, logsm=logsm),
        grid=grid,
        in_specs=[
            pl.BlockSpec((rb, k), lambda i: (i, 0)),
            pl.BlockSpec((k, f), lambda i: (0, 0)),
            pl.BlockSpec((1, f), lambda i: (0, 0)),
            pl.BlockSpec((f, m), lambda i: (0, 0)),
            pl.BlockSpec((1, m), lambda i: (0, 0)),
        ],
        out_specs=pl.BlockSpec((rb, m), lambda i: (i, 0)),
        out_shape=jax.ShapeDtypeStruct((n, m), jnp.float32),
    )(x, w1, b1.reshape(1, f), w2, b2.reshape(1, m))


def _bn_body(x_ref, g_ref, b_ref, o_ref):
    xv = x_ref[...]
    m = jnp.mean(xv, axis=0, keepdims=True)
    v = jnp.mean(jnp.square(xv - m), axis=0, keepdims=True)
    o_ref[...] = g_ref[...] * (xv - m) * jax.lax.rsqrt(v + 1e-5) + b_ref[...]


def _bn(x, g, b):
    n, d = x.shape
    return pl.pallas_call(
        _bn_body,
        out_shape=jax.ShapeDtypeStruct((n, d), jnp.float32),
    )(x, g.reshape(1, d), b.reshape(1, d))


def _gat_sparse(h, src, dst, s_src, s_dst, n):
    a = jax.nn.leaky_relu(s_src[src] + s_dst[dst], 0.2)
    amax = jax.ops.segment_max(a, dst, num_segments=n)
    amax = jnp.where(jnp.isfinite(amax), amax, 0.0)
    ex = jnp.exp(a - amax[dst])
    den = jax.ops.segment_sum(ex, dst, num_segments=n)
    alpha = ex / (den[dst] + 1e-16)
    hs = h[src]
    cols = []
    for hd in range(alpha.shape[1]):
        cols.append(jax.ops.segment_sum(hs * alpha[:, hd:hd + 1], dst,
                                        num_segments=n))
    return jnp.concatenate(cols, axis=1)


def kernel(x, edge_index, params):
    n = x.shape[0]
    src = edge_index[0]
    dst = edge_index[1]

    e = params["emb"]
    h = _mlp2(x, e["W1"], e["b1"], e["W2"], e["b2"])

    for i in (1, 2, 3):
        p = params["enc%d" % i]
        w3 = p["W"].reshape(_D, _HEADS, _D)
        u = jnp.einsum("dhk,hk->dh", w3, p["a_src"][0])
        v = jnp.einsum("dhk,hk->dh", w3, p["a_dst"][0])
        s = _linear(h, jnp.concatenate([u, v], axis=1),
                    jnp.zeros((2 * _HEADS,), jnp.float32))
        t = _gat_sparse(h, src, dst, s[:, :_HEADS], s[:, _HEADS:], n)
        wstack = w3.transpose(1, 0, 2).reshape(_HEADS * _D, _D)
        g = _linear(t, wstack * (1.0 / _HEADS), p["b"])
        f = params["ff%d" % i]
        ff = _mlp2(g + h, f["W1"], f["b1"], f["W2"], f["b2"])
        b = params["bn%d" % i]
        h = _bn(ff + h, b["g"], b["b"])

    pd = params["dec"]
    hd = _linear(h, pd["W"], jnp.zeros((2,), jnp.float32))
    s_src = hd @ pd["a_src"][0].T
    s_dst = hd @ pd["a_dst"][0].T
    a = jax.nn.leaky_relu(s_src[src] + s_dst[dst], 0.2)
    amax = jax.ops.segment_max(a, dst, num_segments=n)
    amax = jnp.where(jnp.isfinite(amax), amax, 0.0)
    ex = jnp.exp(a - amax[dst])
    den = jax.ops.segment_sum(ex, dst, num_segments=n)
    alpha = ex / (den[dst] + 1e-16)
    lp = jax.ops.segment_sum(hd[src] * alpha, dst, num_segments=n) + pd["b"]

    flat = lp.reshape(4, -1)
    pp = params["proj"]
    return _mlp2(flat, pp["W1"], pp["b1"], pp["W2"], pp["b2"],
                 logsm=True, rb=4)


# speedup vs baseline: 4.3978x; 1.4521x over previous
import functools

import jax
import jax.numpy as jnp
from jax import lax
from jax.experimental import pallas as pl
from jax.experimental.pallas import tpu as pltpu
import jax.experimental.pallas.tpu_sc as plsc

_HEADS = 8
_D = 128


def _lin_body(x_ref, w_ref, b_ref, o_ref, *, act):
    y = jnp.dot(x_ref[...], w_ref[...], preferred_element_type=jnp.float32)
    y = y + b_ref[...]
    if act == "relu":
        y = jnp.maximum(y, 0.0)
    o_ref[...] = y


def _linear(x, w, b, act=None, rb=None):
    n, k = x.shape
    m = w.shape[1]
    if rb is None:
        rb = n if n <= 2000 else 1000
    assert n % rb == 0
    grid = (n // rb,)
    return pl.pallas_call(
        functools.partial(_lin_body, act=act),
        grid=grid,
        in_specs=[
            pl.BlockSpec((rb, k), lambda i: (i, 0)),
            pl.BlockSpec((k, m), lambda i: (0, 0)),
            pl.BlockSpec((1, m), lambda i: (0, 0)),
        ],
        out_specs=pl.BlockSpec((rb, m), lambda i: (i, 0)),
        out_shape=jax.ShapeDtypeStruct((n, m), jnp.float32),
    )(x, w, b.reshape(1, m))


def _mlp2_body(x_ref, w1_ref, b1_ref, w2_ref, b2_ref, o_ref, *, logsm):
    mid = jnp.dot(x_ref[...], w1_ref[...], preferred_element_type=jnp.float32)
    mid = jnp.maximum(mid + b1_ref[...], 0.0)
    y = jnp.dot(mid, w2_ref[...], preferred_element_type=jnp.float32)
    y = y + b2_ref[...]
    if logsm:
        y = y - jnp.max(y, axis=1, keepdims=True)
        y = y - jnp.log(jnp.sum(jnp.exp(y), axis=1, keepdims=True))
    o_ref[...] = y


def _mlp2(x, w1, b1, w2, b2, logsm=False, rb=None):
    n, k = x.shape
    f = w1.shape[1]
    m = w2.shape[1]
    if rb is None:
        rb = n if n <= 2000 else 1000
    assert n % rb == 0
    grid = (n // rb,)
    return pl.pallas_call(
        functools.partial(_mlp2_body, logsm=logsm),
        grid=grid,
        in_specs=[
            pl.BlockSpec((rb, k), lambda i: (i, 0)),
            pl.BlockSpec((k, f), lambda i: (0, 0)),
            pl.BlockSpec((1, f), lambda i: (0, 0)),
            pl.BlockSpec((f, m), lambda i: (0, 0)),
            pl.BlockSpec((1, m), lambda i: (0, 0)),
        ],
        out_specs=pl.BlockSpec((rb, m), lambda i: (i, 0)),
        out_shape=jax.ShapeDtypeStruct((n, m), jnp.float32),
    )(x, w1, b1.reshape(1, f), w2, b2.reshape(1, m))


def _ff_body(p0_ref, p1_ref, h_ref, gb_ref, w1_ref, b1_ref, w2_ref, b2_ref,
             o_ref):
    x = p0_ref[...] + p1_ref[...] + h_ref[...] + gb_ref[...]
    mid = jnp.dot(x, w1_ref[...], preferred_element_type=jnp.float32)
    mid = jnp.maximum(mid + b1_ref[...], 0.0)
    y = jnp.dot(mid, w2_ref[...], preferred_element_type=jnp.float32)
    o_ref[...] = y + b2_ref[...]


def _ff_block(p0, p1, h, gb, w1, b1, w2, b2, rb=1000):
    n, k = h.shape
    f = w1.shape[1]
    m = w2.shape[1]
    assert n % rb == 0
    grid = (n // rb,)
    return pl.pallas_call(
        _ff_body,
        grid=grid,
        in_specs=[
            pl.BlockSpec((rb, k), lambda i: (i, 0)),
            pl.BlockSpec((rb, k), lambda i: (i, 0)),
            pl.BlockSpec((rb, k), lambda i: (i, 0)),
            pl.BlockSpec((1, k), lambda i: (0, 0)),
            pl.BlockSpec((k, f), lambda i: (0, 0)),
            pl.BlockSpec((1, f), lambda i: (0, 0)),
            pl.BlockSpec((f, m), lambda i: (0, 0)),
            pl.BlockSpec((1, m), lambda i: (0, 0)),
        ],
        out_specs=pl.BlockSpec((rb, m), lambda i: (i, 0)),
        out_shape=jax.ShapeDtypeStruct((n, m), jnp.float32),
    )(p0, p1, h, gb.reshape(1, k), w1, b1.reshape(1, f), w2,
      b2.reshape(1, m))


def _bn_add_body(x1_ref, x2_ref, g_ref, b_ref, o_ref):
    xv = x1_ref[...] + x2_ref[...]
    m = jnp.mean(xv, axis=0, keepdims=True)
    v = jnp.mean(jnp.square(xv - m), axis=0, keepdims=True)
    o_ref[...] = g_ref[...] * (xv - m) * lax.rsqrt(v + 1e-5) + b_ref[...]


def _bn_add(x1, x2, g, b):
    n, d = x1.shape
    return pl.pallas_call(
        _bn_add_body,
        out_shape=jax.ShapeDtypeStruct((n, d), jnp.float32),
    )(x1, x2, g.reshape(1, d), b.reshape(1, d))


@functools.partial(jax.jit, static_argnames=())
def _gat_aggregate_sc(hh, alpha16, src, dst):
    n = hh.shape[0]
    e = src.shape[0]
    info = plsc.get_sparse_core_info()
    nc, ns = info.num_cores, info.num_subcores
    nw = nc * ns
    K = 16
    SUB = 16
    NSUB = K // SUB
    assert e % nw == 0
    epw = e // nw
    assert epw % K == 0
    n_chunks = epw // K
    ZR = K
    assert n % ZR == 0
    nzc = n // ZR

    mesh = plsc.VectorSubcoreMesh(core_axis_name="c", subcore_axis_name="s")

    @functools.partial(
        pl.kernel, mesh=mesh,
        out_type=jax.ShapeDtypeStruct((nc, n, _D), jnp.float32),
        scratch_types=[
            pltpu.VMEM((2, K), jnp.int32),
            pltpu.VMEM((K // 2, 16), jnp.float32),
            pltpu.VMEM((SUB, _HEADS * _D), jnp.float32),
            pltpu.VMEM((K, _D), jnp.float32),
            pltpu.VMEM_SHARED((n, _D), jnp.float32),
            pltpu.SemaphoreType.DMA,
        ],
    )
    def k(hh_hbm, alpha_hbm, src_hbm, dst_hbm, zeros_hbm, out_hbm,
          idx_v, alpha_v, rows_v, vbuf, acc, sem):
        cid = lax.axis_index("c")
        sid = lax.axis_index("s")

        nmine = (nzc - sid + ns - 1) // ns

        def zchunk(k, carry):
            c = pl.multiple_of((sid + k * ns) * ZR, ZR)
            pltpu.sync_copy(zeros_hbm, acc.at[pl.ds(c, ZR)])
            return carry
        lax.fori_loop(0, nmine, zchunk, 0)
        plsc.subcore_barrier()

        ebase = (cid * ns + sid) * epw

        def chunk(ci, carry):
            eb = pl.multiple_of(ebase + ci * K, K)
            eb2 = pl.multiple_of((ebase + ci * K) // 2, K // 2)
            pltpu.sync_copy(src_hbm.at[pl.ds(eb, K)], idx_v.at[0])
            pltpu.sync_copy(dst_hbm.at[pl.ds(eb, K)], idx_v.at[1])
            pltpu.sync_copy(alpha_hbm.at[pl.ds(eb2, K // 2)], alpha_v)
            for g in range(NSUB):
                pltpu.async_copy(
                    hh_hbm.at[idx_v.at[0, pl.ds(g * SUB, SUB)]],
                    rows_v, sem).wait()

                def edge2(j2, ecarry):
                    av = alpha_v[g * (SUB // 2) + j2, pl.ds(0, 16)]
                    for t in range(2):
                        j = 2 * j2 + t
                        accs = [jnp.zeros((16,), jnp.float32)
                                for _ in range(_D // 16)]
                        for h in range(_HEADS):
                            a = av[t * _HEADS + h]
                            for i in range(_D // 16):
                                accs[i] = accs[i] + a * rows_v[
                                    j, pl.ds(h * _D + i * 16, 16)]
                        for i in range(_D // 16):
                            vbuf[g * SUB + j, pl.ds(i * 16, 16)] = accs[i]
                    return ecarry
                lax.fori_loop(0, SUB // 2, edge2, 0)
            pltpu.sync_copy(vbuf, acc.at[idx_v.at[1]], add=True)
            return carry
        lax.fori_loop(0, n_chunks, chunk, 0)
        plsc.subcore_barrier()

        def ochunk(k, carry):
            c = pl.multiple_of((sid + k * ns) * ZR, ZR)
            pltpu.sync_copy(acc.at[pl.ds(c, ZR)],
                            out_hbm.at[cid, pl.ds(c, ZR)])
            return carry
        lax.fori_loop(0, nmine, ochunk, 0)

    return k(hh, alpha16, src, dst, jnp.zeros((ZR, _D), jnp.float32))


def _edge_alpha(s_src, s_dst, src, dst, n):
    a = jax.nn.leaky_relu(s_src[src] + s_dst[dst], 0.2)
    amax = jax.ops.segment_max(a, dst, num_segments=n)
    amax = jnp.where(jnp.isfinite(amax), amax, 0.0)
    ex = jnp.exp(a - amax[dst])
    den = jax.ops.segment_sum(ex, dst, num_segments=n)
    return ex / (den[dst] + 1e-16)


def kernel(x, edge_index, params):
    n = x.shape[0]
    e = edge_index.shape[1]
    src = edge_index[0]
    dst = edge_index[1]

    emb = params["emb"]
    h = _mlp2(x, emb["W1"], emb["b1"], emb["W2"], emb["b2"])

    for i in (1, 2, 3):
        p = params["enc%d" % i]
        w3 = p["W"].reshape(_D, _HEADS, _D)
        u = jnp.einsum("dhk,hk->dh", w3, p["a_src"][0])
        v = jnp.einsum("dhk,hk->dh", w3, p["a_dst"][0])
        s = _linear(h, jnp.concatenate([u, v], axis=1),
                    jnp.zeros((2 * _HEADS,), jnp.float32))
        alpha = _edge_alpha(s[:, :_HEADS], s[:, _HEADS:], src, dst, n)
        alphap = (alpha * (1.0 / _HEADS)).reshape(e // 2, 2 * _HEADS)
        hh = _linear(h, p["W"], jnp.zeros((_HEADS * _D,), jnp.float32))
        part = _gat_aggregate_sc(hh, alphap, src, dst)
        f = params["ff%d" % i]
        ff = _ff_block(part[0], part[1], h, p["b"],
                       f["W1"], f["b1"], f["W2"], f["b2"])
        b = params["bn%d" % i]
        h = _bn_add(ff, h, b["g"], b["b"])

    pd = params["dec"]
    hd = _linear(h, pd["W"], jnp.zeros((2,), jnp.float32))
    s_src = hd @ pd["a_src"][0].T
    s_dst = hd @ pd["a_dst"][0].T
    alpha_d = _edge_alpha(s_src, s_dst, src, dst, n)
    lp = jax.ops.segment_sum(hd[src] * alpha_d, dst, num_segments=n) + pd["b"]

    flat = lp.reshape(4, -1)
    pp = params["proj"]
    return _mlp2(flat, pp["W1"], pp["b1"], pp["W2"], pp["b2"],
                 logsm=True, rb=4)


# speedup vs baseline: 4.7477x; 1.0796x over previous
import functools

import jax
import jax.numpy as jnp
from jax import lax
from jax.experimental import pallas as pl
from jax.experimental.pallas import tpu as pltpu
import jax.experimental.pallas.tpu_sc as plsc

_HEADS = 8
_D = 128


def _lin_body(x_ref, w_ref, b_ref, o_ref, *, act):
    y = jnp.dot(x_ref[...], w_ref[...], preferred_element_type=jnp.float32)
    y = y + b_ref[...]
    if act == "relu":
        y = jnp.maximum(y, 0.0)
    o_ref[...] = y


def _linear(x, w, b, act=None, rb=None):
    n, k = x.shape
    m = w.shape[1]
    if rb is None:
        rb = n if n <= 2000 else 1000
    assert n % rb == 0
    grid = (n // rb,)
    return pl.pallas_call(
        functools.partial(_lin_body, act=act),
        grid=grid,
        in_specs=[
            pl.BlockSpec((rb, k), lambda i: (i, 0)),
            pl.BlockSpec((k, m), lambda i: (0, 0)),
            pl.BlockSpec((1, m), lambda i: (0, 0)),
        ],
        out_specs=pl.BlockSpec((rb, m), lambda i: (i, 0)),
        out_shape=jax.ShapeDtypeStruct((n, m), jnp.float32),
    )(x, w, b.reshape(1, m))


def _mlp2_body(x_ref, w1_ref, b1_ref, w2_ref, b2_ref, o_ref, *, logsm):
    mid = jnp.dot(x_ref[...], w1_ref[...], preferred_element_type=jnp.float32)
    mid = jnp.maximum(mid + b1_ref[...], 0.0)
    y = jnp.dot(mid, w2_ref[...], preferred_element_type=jnp.float32)
    y = y + b2_ref[...]
    if logsm:
        y = y - jnp.max(y, axis=1, keepdims=True)
        y = y - jnp.log(jnp.sum(jnp.exp(y), axis=1, keepdims=True))
    o_ref[...] = y


def _mlp2(x, w1, b1, w2, b2, logsm=False, rb=None):
    n, k = x.shape
    f = w1.shape[1]
    m = w2.shape[1]
    if rb is None:
        rb = n if n <= 2000 else 1000
    assert n % rb == 0
    grid = (n // rb,)
    return pl.pallas_call(
        functools.partial(_mlp2_body, logsm=logsm),
        grid=grid,
        in_specs=[
            pl.BlockSpec((rb, k), lambda i: (i, 0)),
            pl.BlockSpec((k, f), lambda i: (0, 0)),
            pl.BlockSpec((1, f), lambda i: (0, 0)),
            pl.BlockSpec((f, m), lambda i: (0, 0)),
            pl.BlockSpec((1, m), lambda i: (0, 0)),
        ],
        out_specs=pl.BlockSpec((rb, m), lambda i: (i, 0)),
        out_shape=jax.ShapeDtypeStruct((n, m), jnp.float32),
    )(x, w1, b1.reshape(1, f), w2, b2.reshape(1, m))


def _ff_body(p0_ref, p1_ref, h_ref, gb_ref, w1_ref, b1_ref, w2_ref, b2_ref,
             o_ref):
    x = p0_ref[...] + p1_ref[...] + h_ref[...] + gb_ref[...]
    mid = jnp.dot(x, w1_ref[...], preferred_element_type=jnp.float32)
    mid = jnp.maximum(mid + b1_ref[...], 0.0)
    y = jnp.dot(mid, w2_ref[...], preferred_element_type=jnp.float32)
    o_ref[...] = y + b2_ref[...]


def _ff_block(p0, p1, h, gb, w1, b1, w2, b2, rb=1000):
    n, k = h.shape
    f = w1.shape[1]
    m = w2.shape[1]
    assert n % rb == 0
    grid = (n // rb,)
    return pl.pallas_call(
        _ff_body,
        grid=grid,
        in_specs=[
            pl.BlockSpec((rb, k), lambda i: (i, 0)),
            pl.BlockSpec((rb, k), lambda i: (i, 0)),
            pl.BlockSpec((rb, k), lambda i: (i, 0)),
            pl.BlockSpec((1, k), lambda i: (0, 0)),
            pl.BlockSpec((k, f), lambda i: (0, 0)),
            pl.BlockSpec((1, f), lambda i: (0, 0)),
            pl.BlockSpec((f, m), lambda i: (0, 0)),
            pl.BlockSpec((1, m), lambda i: (0, 0)),
        ],
        out_specs=pl.BlockSpec((rb, m), lambda i: (i, 0)),
        out_shape=jax.ShapeDtypeStruct((n, m), jnp.float32),
    )(p0, p1, h, gb.reshape(1, k), w1, b1.reshape(1, f), w2,
      b2.reshape(1, m))


def _bn_add_body(x1_ref, x2_ref, g_ref, b_ref, o_ref):
    xv = x1_ref[...] + x2_ref[...]
    m = jnp.mean(xv, axis=0, keepdims=True)
    v = jnp.mean(jnp.square(xv - m), axis=0, keepdims=True)
    o_ref[...] = g_ref[...] * (xv - m) * lax.rsqrt(v + 1e-5) + b_ref[...]


def _bn_add(x1, x2, g, b):
    n, d = x1.shape
    return pl.pallas_call(
        _bn_add_body,
        out_shape=jax.ShapeDtypeStruct((n, d), jnp.float32),
    )(x1, x2, g.reshape(1, d), b.reshape(1, d))


@functools.partial(jax.jit, static_argnames=())
def _gat_aggregate_sc(hh, alpha16, src, dst):
    n = hh.shape[0]
    e = src.shape[0]
    info = plsc.get_sparse_core_info()
    nc, ns = info.num_cores, info.num_subcores
    nw = nc * ns
    K = 16
    SUB = 16
    NSUB = K // SUB
    assert e % nw == 0
    epw = e // nw
    assert epw % K == 0
    n_chunks = epw // K
    ZR = K
    assert n % ZR == 0
    nzc = n // ZR

    mesh = plsc.VectorSubcoreMesh(core_axis_name="c", subcore_axis_name="s")

    @functools.partial(
        pl.kernel, mesh=mesh,
        out_type=jax.ShapeDtypeStruct((nc, n, _D), jnp.float32),
        scratch_types=[
            pltpu.VMEM((2, K), jnp.int32),
            pltpu.VMEM((K // 2, 16), jnp.float32),
            pltpu.VMEM((SUB, _HEADS * _D), jnp.float32),
            pltpu.VMEM((K, _D), jnp.float32),
            pltpu.VMEM_SHARED((n, _D), jnp.float32),
            pltpu.SemaphoreType.DMA,
        ],
    )
    def k(hh_hbm, alpha_hbm, src_hbm, dst_hbm, zeros_hbm, out_hbm,
          idx_v, alpha_v, rows_v, vbuf, acc, sem):
        cid = lax.axis_index("c")
        sid = lax.axis_index("s")

        nmine = (nzc - sid + ns - 1) // ns

        def zchunk(k, carry):
            c = pl.multiple_of((sid + k * ns) * ZR, ZR)
            pltpu.sync_copy(zeros_hbm, acc.at[pl.ds(c, ZR)])
            return carry
        lax.fori_loop(0, nmine, zchunk, 0)
        plsc.subcore_barrier()

        ebase = (cid * ns + sid) * epw

        def chunk(ci, carry):
            eb = pl.multiple_of(ebase + ci * K, K)
            eb2 = pl.multiple_of((ebase + ci * K) // 2, K // 2)
            pltpu.sync_copy(src_hbm.at[pl.ds(eb, K)], idx_v.at[0])
            pltpu.sync_copy(dst_hbm.at[pl.ds(eb, K)], idx_v.at[1])
            pltpu.sync_copy(alpha_hbm.at[pl.ds(eb2, K // 2)], alpha_v)
            for g in range(NSUB):
                pltpu.async_copy(
                    hh_hbm.at[idx_v.at[0, pl.ds(g * SUB, SUB)]],
                    rows_v, sem).wait()

                def edge2(j2, ecarry):
                    av = alpha_v[g * (SUB // 2) + j2, pl.ds(0, 16)]
                    for t in range(2):
                        j = 2 * j2 + t
                        accs = [jnp.zeros((16,), jnp.float32)
                                for _ in range(_D // 16)]
                        for h in range(_HEADS):
                            a = av[t * _HEADS + h]
                            for i in range(_D // 16):
                                accs[i] = accs[i] + a * rows_v[
                                    j, pl.ds(h * _D + i * 16, 16)]
                        for i in range(_D // 16):
                            vbuf[g * SUB + j, pl.ds(i * 16, 16)] = accs[i]
                    return ecarry
                lax.fori_loop(0, SUB // 2, edge2, 0)
            pltpu.sync_copy(vbuf, acc.at[idx_v.at[1]], add=True)
            return carry
        lax.fori_loop(0, n_chunks, chunk, 0)
        plsc.subcore_barrier()

        def ochunk(k, carry):
            c = pl.multiple_of((sid + k * ns) * ZR, ZR)
            pltpu.sync_copy(acc.at[pl.ds(c, ZR)],
                            out_hbm.at[cid, pl.ds(c, ZR)])
            return carry
        lax.fori_loop(0, nmine, ochunk, 0)

    return k(hh, alpha16, src, dst, jnp.zeros((ZR, _D), jnp.float32))


def _edge_alpha(s_src, s_dst, src, dst, n):
    gmax = jnp.max(s_src, axis=0, keepdims=True)
    c = jax.nn.leaky_relu(gmax + s_dst, 0.2)
    a = jax.nn.leaky_relu(s_src[src] + s_dst[dst], 0.2)
    ex = jnp.exp(a - c[dst])
    den = jax.ops.segment_sum(ex, dst, num_segments=n)
    return ex / (den[dst] + 1e-16)


def kernel(x, edge_index, params):
    n = x.shape[0]
    e = edge_index.shape[1]
    src = edge_index[0]
    dst = edge_index[1]

    emb = params["emb"]
    h = _mlp2(x, emb["W1"], emb["b1"], emb["W2"], emb["b2"])

    for i in (1, 2, 3):
        p = params["enc%d" % i]
        w3 = p["W"].reshape(_D, _HEADS, _D)
        u = jnp.einsum("dhk,hk->dh", w3, p["a_src"][0])
        v = jnp.einsum("dhk,hk->dh", w3, p["a_dst"][0])
        s = _linear(h, jnp.concatenate([u, v], axis=1),
                    jnp.zeros((2 * _HEADS,), jnp.float32))
        alpha = _edge_alpha(s[:, :_HEADS], s[:, _HEADS:], src, dst, n)
        alphap = (alpha * (1.0 / _HEADS)).reshape(e // 2, 2 * _HEADS)
        hh = _linear(h, p["W"], jnp.zeros((_HEADS * _D,), jnp.float32))
        part = _gat_aggregate_sc(hh, alphap, src, dst)
        f = params["ff%d" % i]
        ff = _ff_block(part[0], part[1], h, p["b"],
                       f["W1"], f["b1"], f["W2"], f["b2"])
        b = params["bn%d" % i]
        h = _bn_add(ff, h, b["g"], b["b"])

    pd = params["dec"]
    hd = _linear(h, pd["W"], jnp.zeros((2,), jnp.float32))
    s_src = hd @ pd["a_src"][0].T
    s_dst = hd @ pd["a_dst"][0].T
    alpha_d = _edge_alpha(s_src, s_dst, src, dst, n)
    lp = jax.ops.segment_sum(hd[src] * alpha_d, dst, num_segments=n) + pd["b"]

    flat = lp.reshape(4, -1)
    pp = params["proj"]
    return _mlp2(flat, pp["W1"], pp["b1"], pp["W2"], pp["b2"],
                 logsm=True, rb=4)


# speedup vs baseline: 5.0081x; 1.0549x over previous
import functools

import jax
import jax.numpy as jnp
from jax import lax
from jax.experimental import pallas as pl
from jax.experimental.pallas import tpu as pltpu
import jax.experimental.pallas.tpu_sc as plsc

_HEADS = 8
_D = 128


def _lin_body(x_ref, w_ref, b_ref, o_ref, *, act):
    y = jnp.dot(x_ref[...], w_ref[...], preferred_element_type=jnp.float32)
    y = y + b_ref[...]
    if act == "relu":
        y = jnp.maximum(y, 0.0)
    o_ref[...] = y


def _linear(x, w, b, act=None, rb=None):
    n, k = x.shape
    m = w.shape[1]
    if rb is None:
        rb = n if n <= 2000 else 1000
    assert n % rb == 0
    grid = (n // rb,)
    return pl.pallas_call(
        functools.partial(_lin_body, act=act),
        grid=grid,
        in_specs=[
            pl.BlockSpec((rb, k), lambda i: (i, 0)),
            pl.BlockSpec((k, m), lambda i: (0, 0)),
            pl.BlockSpec((1, m), lambda i: (0, 0)),
        ],
        out_specs=pl.BlockSpec((rb, m), lambda i: (i, 0)),
        out_shape=jax.ShapeDtypeStruct((n, m), jnp.float32),
    )(x, w, b.reshape(1, m))


def _mlp2_body(x_ref, w1_ref, b1_ref, w2_ref, b2_ref, o_ref, *, logsm):
    mid = jnp.dot(x_ref[...], w1_ref[...], preferred_element_type=jnp.float32)
    mid = jnp.maximum(mid + b1_ref[...], 0.0)
    y = jnp.dot(mid, w2_ref[...], preferred_element_type=jnp.float32)
    y = y + b2_ref[...]
    if logsm:
        y = y - jnp.max(y, axis=1, keepdims=True)
        y = y - jnp.log(jnp.sum(jnp.exp(y), axis=1, keepdims=True))
    o_ref[...] = y


def _mlp2(x, w1, b1, w2, b2, logsm=False, rb=None):
    n, k = x.shape
    f = w1.shape[1]
    m = w2.shape[1]
    if rb is None:
        rb = n if n <= 2000 else 1000
    assert n % rb == 0
    grid = (n // rb,)
    return pl.pallas_call(
        functools.partial(_mlp2_body, logsm=logsm),
        grid=grid,
        in_specs=[
            pl.BlockSpec((rb, k), lambda i: (i, 0)),
            pl.BlockSpec((k, f), lambda i: (0, 0)),
            pl.BlockSpec((1, f), lambda i: (0, 0)),
            pl.BlockSpec((f, m), lambda i: (0, 0)),
            pl.BlockSpec((1, m), lambda i: (0, 0)),
        ],
        out_specs=pl.BlockSpec((rb, m), lambda i: (i, 0)),
        out_shape=jax.ShapeDtypeStruct((n, m), jnp.float32),
    )(x, w1, b1.reshape(1, f), w2, b2.reshape(1, m))


def _ff_body(p0_ref, p1_ref, h_ref, gb_ref, w1_ref, b1_ref, w2_ref, b2_ref,
             o_ref):
    x = p0_ref[...] + p1_ref[...] + h_ref[...] + gb_ref[...]
    mid = jnp.dot(x, w1_ref[...], preferred_element_type=jnp.float32)
    mid = jnp.maximum(mid + b1_ref[...], 0.0)
    y = jnp.dot(mid, w2_ref[...], preferred_element_type=jnp.float32)
    o_ref[...] = y + b2_ref[...]


def _ff_block(p0, p1, h, gb, w1, b1, w2, b2, rb=1000):
    n, k = h.shape
    f = w1.shape[1]
    m = w2.shape[1]
    assert n % rb == 0
    grid = (n // rb,)
    return pl.pallas_call(
        _ff_body,
        grid=grid,
        in_specs=[
            pl.BlockSpec((rb, k), lambda i: (i, 0)),
            pl.BlockSpec((rb, k), lambda i: (i, 0)),
            pl.BlockSpec((rb, k), lambda i: (i, 0)),
            pl.BlockSpec((1, k), lambda i: (0, 0)),
            pl.BlockSpec((k, f), lambda i: (0, 0)),
            pl.BlockSpec((1, f), lambda i: (0, 0)),
            pl.BlockSpec((f, m), lambda i: (0, 0)),
            pl.BlockSpec((1, m), lambda i: (0, 0)),
        ],
        out_specs=pl.BlockSpec((rb, m), lambda i: (i, 0)),
        out_shape=jax.ShapeDtypeStruct((n, m), jnp.float32),
    )(p0, p1, h, gb.reshape(1, k), w1, b1.reshape(1, f), w2,
      b2.reshape(1, m))


def _bn_add_body(x1_ref, x2_ref, g_ref, b_ref, o_ref):
    xv = x1_ref[...] + x2_ref[...]
    m = jnp.mean(xv, axis=0, keepdims=True)
    v = jnp.mean(jnp.square(xv - m), axis=0, keepdims=True)
    o_ref[...] = g_ref[...] * (xv - m) * lax.rsqrt(v + 1e-5) + b_ref[...]


def _bn_add(x1, x2, g, b):
    n, d = x1.shape
    return pl.pallas_call(
        _bn_add_body,
        out_shape=jax.ShapeDtypeStruct((n, d), jnp.float32),
    )(x1, x2, g.reshape(1, d), b.reshape(1, d))


@functools.partial(jax.jit, static_argnames=())
def _gat_aggregate_sc(hh, alpha16, src, dst):
    n = hh.shape[0]
    e = src.shape[0]
    info = plsc.get_sparse_core_info()
    nc, ns = info.num_cores, info.num_subcores
    nw = nc * ns
    K = 80
    SUB = 16
    NSUB = K // SUB
    SC = 16
    NSC = K // SC
    assert e % nw == 0
    epw = e // nw
    assert epw % K == 0
    n_chunks = epw // K
    ZR = K
    assert n % ZR == 0
    nzc = n // ZR

    mesh = plsc.VectorSubcoreMesh(core_axis_name="c", subcore_axis_name="s")

    @functools.partial(
        pl.kernel, mesh=mesh,
        out_type=jax.ShapeDtypeStruct((nc, n, _D), jnp.float32),
        scratch_types=[
            pltpu.VMEM((K,), jnp.int32),
            pltpu.VMEM((NSC, SC), jnp.int32),
            pltpu.VMEM((K // 2, 16), jnp.float32),
            pltpu.VMEM((SUB, _HEADS * _D), jnp.float32),
            pltpu.VMEM((K, _D), jnp.float32),
            pltpu.VMEM_SHARED((n, _D), jnp.float32),
            pltpu.SemaphoreType.DMA,
        ],
    )
    def k(hh_hbm, alpha_hbm, src_hbm, dst_hbm, zeros_hbm, out_hbm,
          src_v, dst_v, alpha_v, rows_v, vbuf, acc, sem):
        cid = lax.axis_index("c")
        sid = lax.axis_index("s")

        nmine = (nzc - sid + ns - 1) // ns

        def zchunk(k, carry):
            c = pl.multiple_of((sid + k * ns) * ZR, ZR)
            pltpu.sync_copy(zeros_hbm, acc.at[pl.ds(c, ZR)])
            return carry
        lax.fori_loop(0, nmine, zchunk, 0)
        plsc.subcore_barrier()

        ebase = (cid * ns + sid) * epw

        def chunk(ci, carry):
            eb = pl.multiple_of(ebase + ci * K, K)
            eb2 = pl.multiple_of((ebase + ci * K) // 2, K // 2)
            pltpu.sync_copy(src_hbm.at[pl.ds(eb, K)], src_v)
            for g2 in range(NSC):
                pltpu.sync_copy(dst_hbm.at[pl.ds(eb + g2 * SC, SC)],
                                dst_v.at[g2])
            pltpu.sync_copy(alpha_hbm.at[pl.ds(eb2, K // 2)], alpha_v)
            for g in range(NSUB):
                pltpu.async_copy(
                    hh_hbm.at[src_v.at[pl.ds(g * SUB, SUB)]],
                    rows_v, sem).wait()

                def edge2(j2, ecarry):
                    av = alpha_v[g * (SUB // 2) + j2, pl.ds(0, 16)]
                    for t in range(2):
                        j = 2 * j2 + t
                        accs = [jnp.zeros((16,), jnp.float32)
                                for _ in range(_D // 16)]
                        for h in range(_HEADS):
                            a = av[t * _HEADS + h]
                            for i in range(_D // 16):
                                accs[i] = accs[i] + a * rows_v[
                                    j, pl.ds(h * _D + i * 16, 16)]
                        for i in range(_D // 16):
                            vbuf[g * SUB + j, pl.ds(i * 16, 16)] = accs[i]
                    return ecarry
                lax.fori_loop(0, SUB // 2, edge2, 0)
            for g2 in range(NSC):
                pltpu.sync_copy(vbuf.at[pl.ds(g2 * SC, SC)],
                                acc.at[dst_v.at[g2]], add=True)
            return carry
        lax.fori_loop(0, n_chunks, chunk, 0)
        plsc.subcore_barrier()

        def ochunk(k, carry):
            c = pl.multiple_of((sid + k * ns) * ZR, ZR)
            pltpu.sync_copy(acc.at[pl.ds(c, ZR)],
                            out_hbm.at[cid, pl.ds(c, ZR)])
            return carry
        lax.fori_loop(0, nmine, ochunk, 0)

    return k(hh, alpha16, src, dst, jnp.zeros((ZR, _D), jnp.float32))


def _edge_alpha(s_src, s_dst, src, dst, n):
    gmax = jnp.max(s_src, axis=0, keepdims=True)
    c = jax.nn.leaky_relu(gmax + s_dst, 0.2)
    a = jax.nn.leaky_relu(s_src[src] + s_dst[dst], 0.2)
    ex = jnp.exp(a - c[dst])
    den = jax.ops.segment_sum(ex, dst, num_segments=n)
    return ex / (den[dst] + 1e-16)


def kernel(x, edge_index, params):
    n = x.shape[0]
    e = edge_index.shape[1]
    src = edge_index[0]
    dst = edge_index[1]

    emb = params["emb"]
    h = _mlp2(x, emb["W1"], emb["b1"], emb["W2"], emb["b2"])

    for i in (1, 2, 3):
        p = params["enc%d" % i]
        w3 = p["W"].reshape(_D, _HEADS, _D)
        u = jnp.einsum("dhk,hk->dh", w3, p["a_src"][0])
        v = jnp.einsum("dhk,hk->dh", w3, p["a_dst"][0])
        s = _linear(h, jnp.concatenate([u, v], axis=1),
                    jnp.zeros((2 * _HEADS,), jnp.float32))
        alpha = _edge_alpha(s[:, :_HEADS], s[:, _HEADS:], src, dst, n)
        alphap = (alpha * (1.0 / _HEADS)).reshape(e // 2, 2 * _HEADS)
        hh = _linear(h, p["W"], jnp.zeros((_HEADS * _D,), jnp.float32))
        part = _gat_aggregate_sc(hh, alphap, src, dst)
        f = params["ff%d" % i]
        ff = _ff_block(part[0], part[1], h, p["b"],
                       f["W1"], f["b1"], f["W2"], f["b2"])
        b = params["bn%d" % i]
        h = _bn_add(ff, h, b["g"], b["b"])

    pd = params["dec"]
    hd = _linear(h, pd["W"], jnp.zeros((2,), jnp.float32))
    s_src = hd @ pd["a_src"][0].T
    s_dst = hd @ pd["a_dst"][0].T
    alpha_d = _edge_alpha(s_src, s_dst, src, dst, n)
    lp = jax.ops.segment_sum(hd[src] * alpha_d, dst, num_segments=n) + pd["b"]

    flat = lp.reshape(4, -1)
    pp = params["proj"]
    return _mlp2(flat, pp["W1"], pp["b1"], pp["W2"], pp["b2"],
                 logsm=True, rb=4)
